# baseline (speedup 1.0000x reference)
import functools

import jax
import jax.numpy as jnp
from jax import lax
from jax.experimental import pallas as pl
from jax.experimental.pallas import tpu as pltpu
from jax.experimental.pallas import tpu_sc as plsc

B = 4096
H = 50
D = 64
L = 16
NC = 2
NS = 16
NW = NC * NS
RPW = B // NW


def _body(ids_hbm, table_hbm, out_hbm, ids_v, idxT_v, acc_v, sem, ids_sem):
    wid = lax.axis_index("s") * NC + lax.axis_index("c")
    base = wid * RPW

    ids_cp = pltpu.make_async_copy(ids_hbm.at[pl.ds(base, RPW)], ids_v, ids_sem)
    ids_cp.start()

    zero = jnp.zeros((L,), jnp.float32)

    def zbody(i, _):
        for d in range(D // L):
            acc_v[i, pl.ds(d * L, L)] = zero
        return 0

    lax.fori_loop(0, RPW, zbody, 0)
    ids_cp.wait()

    iota = lax.iota(jnp.int32, L)

    def tbody(h, _):
        cols = jnp.zeros((L,), jnp.int32) + h
        for j in range(RPW // L):
            rows = iota + (j * L)
            idxT_v[h, pl.ds(j * L, L)] = plsc.load_gather(ids_v, [rows, cols])
        return 0

    lax.fori_loop(0, H, tbody, 0)

    copies = [
        pltpu.async_copy(table_hbm.at[idxT_v.at[h]], acc_v, sem, add=True)
        for h in range(H)
    ]
    for c in copies:
        c.wait()

    pltpu.sync_copy(acc_v, out_hbm.at[pl.ds(base, RPW)])


_embed_pool = functools.partial(
    pl.kernel,
    out_type=jax.ShapeDtypeStruct((B, D), jnp.float32),
    mesh=plsc.VectorSubcoreMesh(core_axis_name="c", subcore_axis_name="s"),
    scratch_types=[
        pltpu.VMEM((RPW, H), jnp.int32),
        pltpu.VMEM((H, RPW), jnp.int32),
        pltpu.VMEM((RPW, D), jnp.float32),
        pltpu.SemaphoreType.DMA,
        pltpu.SemaphoreType.DMA,
    ],
    compiler_params=pltpu.CompilerParams(
        use_tc_tiling_on_sc=False, needs_layout_passes=False
    ),
)(_body)


def kernel(ids, table):
    return _embed_pool(ids.astype(jnp.int32), table)


# speedup vs baseline: 1.0113x; 1.0113x over previous
import functools

import jax
import jax.numpy as jnp
from jax import lax
from jax.experimental import pallas as pl
from jax.experimental.pallas import tpu as pltpu
from jax.experimental.pallas import tpu_sc as plsc

B = 4096
H = 50
D = 64
L = 16
NC = 2
NS = 16
NW = NC * NS
RPW = B // NW


def _body(idsT_hbm, table_hbm, out_hbm, idxT_v, acc_v, sem, ids_sem):
    wid = lax.axis_index("s") * NC + lax.axis_index("c")
    base = wid * RPW

    ids_cp = pltpu.make_async_copy(
        idsT_hbm.at[:, pl.ds(base, RPW)], idxT_v, ids_sem
    )
    ids_cp.start()

    zero = jnp.zeros((L,), jnp.float32)

    def zbody(i, _):
        for d in range(D // L):
            acc_v[i, pl.ds(d * L, L)] = zero
        return 0

    lax.fori_loop(0, RPW, zbody, 0)
    ids_cp.wait()

    copies = [
        pltpu.async_copy(table_hbm.at[idxT_v.at[h]], acc_v, sem, add=True)
        for h in range(H)
    ]
    for c in copies:
        c.wait()

    pltpu.sync_copy(acc_v, out_hbm.at[pl.ds(base, RPW)])


_embed_pool = functools.partial(
    pl.kernel,
    out_type=jax.ShapeDtypeStruct((B, D), jnp.float32),
    mesh=plsc.VectorSubcoreMesh(core_axis_name="c", subcore_axis_name="s"),
    scratch_types=[
        pltpu.VMEM((H, RPW), jnp.int32),
        pltpu.VMEM((RPW, D), jnp.float32),
        pltpu.SemaphoreType.DMA,
        pltpu.SemaphoreType.DMA,
    ],
    compiler_params=pltpu.CompilerParams(
        use_tc_tiling_on_sc=False, needs_layout_passes=False
    ),
)(_body)


def kernel(ids, table):
    return _embed_pool(ids.astype(jnp.int32).T, table)
